# baseline (speedup 1.0000x reference)
import functools

import jax
import jax.numpy as jnp
from jax.experimental import pallas as pl
from jax.experimental.pallas import tpu as pltpu

NBITS = 32
TOPK = 32
KB = 1024
GW = 6


def _dot(a, b, prec=None):
    return jax.lax.dot_general(a, b, (((1,), (0,)), ((), ())),
                               precision=prec,
                               preferred_element_type=jnp.float32)


def _body(nblk, k_real, q_ref, r_ref, m_ref, tv_ref, y_ref,
          qc_ref, mc_ref, u_ref, ones_ref, cumA_ref, vB_ref, cumB_ref,
          jc_ref, cumbef_ref, vt_ref, rr_ref, runcnt_ref, acc_ref):
    p = pl.program_id(0)
    i = pl.program_id(1)
    qshape = qc_ref.shape[0]
    hi = jax.lax.Precision.HIGHEST

    @pl.when((p == 0) & (i == 0))
    def _init():
        proj_q = jax.lax.dot_general(
            q_ref[...].astype(jnp.bfloat16), r_ref[...].astype(jnp.bfloat16),
            (((1,), (0,)), ((), ())), preferred_element_type=jnp.float32)
        qc_ref[...] = jnp.where(proj_q > 0, 1.0, -1.0).astype(jnp.bfloat16)
        cumA_ref[...] = jnp.zeros_like(cumA_ref)
        rowi = jax.lax.broadcasted_iota(jnp.int32, (KB, KB), 0)
        coli = jax.lax.broadcasted_iota(jnp.int32, (KB, KB), 1)
        u_ref[...] = (rowi <= coli).astype(jnp.bfloat16)
        ones_ref[...] = jnp.ones_like(ones_ref)

    @pl.when(p == 0)
    def _codes():
        proj_m = jax.lax.dot_general(
            m_ref[...].astype(jnp.bfloat16), r_ref[...].astype(jnp.bfloat16),
            (((1,), (0,)), ((), ())), preferred_element_type=jnp.float32)
        mc_ref[pl.ds(i * KB, KB), :] = jnp.where(
            proj_m > 0, 1.0, -1.0).astype(jnp.bfloat16)

    mc = mc_ref[pl.ds(i * KB, KB), :]
    sim = jax.lax.dot_general(
        qc_ref[...], mc, (((1,), (1,)), ((), ())),
        preferred_element_type=jnp.float32)
    col = jax.lax.broadcasted_iota(jnp.int32, (1, KB), 1) + i * KB
    sim = jnp.where(col < k_real, sim, -100.0)
    simb = sim.astype(jnp.bfloat16)

    @pl.when(p == 0)
    def _coarse():
        cols = []
        for j in range(5):
            v = float(NBITS - 2 * (GW * j + GW - 1))
            mask = (simb >= jnp.bfloat16(v)).astype(jnp.bfloat16)
            cols.append(_dot(mask, ones_ref[...]))
        cumA_ref[...] += jnp.concatenate(cols, axis=1)

    @pl.when((p == 0) & (i == nblk - 1))
    def _coarse_combine():
        cumA = cumA_ref[...]
        ge = jnp.where(cumA >= float(TOPK), 1.0, 0.0)
        jc = 5.0 - jnp.sum(ge, axis=1, keepdims=True)
        binj = jax.lax.broadcasted_iota(jnp.int32, cumA.shape, 1).astype(
            jnp.float32)
        cumbef_ref[...] = jnp.sum(
            jnp.where(binj == jc - 1.0, cumA, 0.0), axis=1, keepdims=True)
        cum_top = jnp.sum(
            jnp.where(binj == jc, cumA, 0.0), axis=1, keepdims=True)
        cum_top = jnp.where(jc == 5.0, float(k_real), cum_top)
        jc_ref[...] = jc
        cumB_ref[...] = jnp.concatenate(
            [jnp.zeros((qshape, 5), jnp.float32), cum_top], axis=1)
        vcols = []
        for d in range(5):
            b_d = jnp.minimum(float(GW) * jc + float(d), float(NBITS))
            vcols.append(float(NBITS) - 2.0 * b_d)
        vB_ref[...] = jnp.concatenate(vcols, axis=1)

    @pl.when(p == 1)
    def _fine():
        vBb = vB_ref[...].astype(jnp.bfloat16)
        cols = []
        for d in range(5):
            mask = (simb >= vBb[:, d:d + 1]).astype(jnp.bfloat16)
            cols.append(_dot(mask, ones_ref[...]))
        cumB_ref[:, :5] += jnp.concatenate(cols, axis=1)

    @pl.when((p == 1) & (i == nblk - 1))
    def _fine_combine():
        cumB = cumB_ref[...]
        ge = jnp.where(cumB >= float(TOPK), 1.0, 0.0)
        dstar = 6.0 - jnp.sum(ge, axis=1, keepdims=True)
        t = jnp.minimum(float(GW) * jc_ref[...] + dstar, float(NBITS))
        bind = jax.lax.broadcasted_iota(jnp.int32, cumB.shape, 1).astype(
            jnp.float32)
        prevcum = jnp.sum(
            jnp.where(bind == dstar - 1.0, cumB, 0.0), axis=1, keepdims=True)
        prevcum = prevcum + jnp.where(dstar == 0.0, cumbef_ref[...], 0.0)
        rr_ref[...] = float(TOPK) - prevcum
        vt_ref[...] = float(NBITS) - 2.0 * t
        runcnt_ref[...] = jnp.zeros_like(runcnt_ref)
        acc_ref[...] = jnp.zeros_like(acc_ref)

    @pl.when(p == 2)
    def _sums():
        tvc = tv_ref[...]
        vt = vt_ref[...]
        gtf = (sim > vt).astype(jnp.float32)
        eqf = (sim == vt).astype(jnp.float32)
        eqb = eqf.astype(jnp.bfloat16)
        pre = _dot(eqb, u_ref[...])
        rhs = rr_ref[...] - runcnt_ref[...]
        self_f = jnp.where(pre <= rhs, eqf, 0.0)
        acc_ref[...] += _dot(gtf, tvc, hi) + _dot(self_f, tvc, hi)
        runcnt_ref[...] += _dot(eqb, ones_ref[...])

    @pl.when((p == 2) & (i == nblk - 1))
    def _finalize():
        y_ref[...] = acc_ref[...] * (1.0 / float(TOPK))


@jax.jit
def kernel(h_query, memory_embeds, true_values, R):
    q, d2 = h_query.shape
    k_real = memory_embeds.shape[0]
    nblk = (k_real + KB - 1) // KB
    k_pad = nblk * KB
    if k_pad != k_real:
        memory_embeds = jnp.pad(memory_embeds, ((0, k_pad - k_real), (0, 0)))
        true_values = jnp.pad(true_values, (0, k_pad - k_real))
    tvcol = true_values.reshape(k_pad, 1)

    y = pl.pallas_call(
        functools.partial(_body, nblk, k_real),
        grid=(3, nblk),
        in_specs=[
            pl.BlockSpec((q, d2), lambda p, i: (0, 0)),
            pl.BlockSpec((d2, NBITS), lambda p, i: (0, 0)),
            pl.BlockSpec((KB, d2), lambda p, i: (i, 0)),
            pl.BlockSpec((KB, 1), lambda p, i: (i, 0)),
        ],
        out_specs=pl.BlockSpec((q, 1), lambda p, i: (0, 0)),
        out_shape=jax.ShapeDtypeStruct((q, 1), jnp.float32),
        scratch_shapes=[
            pltpu.VMEM((q, NBITS), jnp.bfloat16),
            pltpu.VMEM((k_pad, NBITS), jnp.bfloat16),
            pltpu.VMEM((KB, KB), jnp.bfloat16),
            pltpu.VMEM((KB, 1), jnp.bfloat16),
            pltpu.VMEM((q, 5), jnp.float32),
            pltpu.VMEM((q, 5), jnp.float32),
            pltpu.VMEM((q, 6), jnp.float32),
            pltpu.VMEM((q, 1), jnp.float32),
            pltpu.VMEM((q, 1), jnp.float32),
            pltpu.VMEM((q, 1), jnp.float32),
            pltpu.VMEM((q, 1), jnp.float32),
            pltpu.VMEM((q, 1), jnp.float32),
            pltpu.VMEM((q, 1), jnp.float32),
        ],
        compiler_params=pltpu.CompilerParams(
            dimension_semantics=("arbitrary", "arbitrary")),
    )(h_query, R, memory_embeds, tvcol)
    return y[:, 0]


# speedup vs baseline: 1.5210x; 1.5210x over previous
import functools

import jax
import jax.numpy as jnp
from jax.experimental import pallas as pl
from jax.experimental.pallas import tpu as pltpu

NBITS = 32
TOPK = 32
KB = 1024
CKB = 128
NCOL = NBITS + 1


def _body(nblk, k_real, q_ref, r_ref, m_ref, tv_ref, y_ref,
          qc_ref, mc_ref, u_ref, cb_ref, vt_ref, rr_ref):
    i = pl.program_id(0)
    qshape = qc_ref.shape[0]

    def block_sim(c, width):
        mc = mc_ref[pl.ds(c * width, width), :]
        return jax.lax.dot_general(
            qc_ref[...], mc, (((1,), (1,)), ((), ())),
            preferred_element_type=jnp.float32)

    @pl.when(i == 0)
    def _init():
        proj_q = jax.lax.dot_general(
            q_ref[...].astype(jnp.bfloat16), r_ref[...].astype(jnp.bfloat16),
            (((1,), (0,)), ((), ())), preferred_element_type=jnp.float32)
        proj_q = jnp.pad(proj_q, ((0, 0), (0, 1)), constant_values=1.0)
        qc_ref[...] = jnp.where(proj_q > 0, 1.0, -1.0).astype(jnp.bfloat16)
        cb_ref[...] = jnp.zeros_like(cb_ref)
        rowi = jax.lax.broadcasted_iota(jnp.int32, (CKB, CKB), 0)
        coli = jax.lax.broadcasted_iota(jnp.int32, (CKB, CKB), 1)
        u_ref[...] = (rowi <= coli).astype(jnp.bfloat16)

    @pl.when(i < nblk)
    def _phase_a():
        proj_m = jax.lax.dot_general(
            m_ref[...].astype(jnp.bfloat16), r_ref[...].astype(jnp.bfloat16),
            (((1,), (0,)), ((), ())), preferred_element_type=jnp.float32)
        codes = jnp.where(proj_m > 0, 1.0, -1.0).astype(jnp.bfloat16)
        rowg = jax.lax.broadcasted_iota(jnp.int32, (KB, 1), 0) + i * KB
        guard = jnp.where(rowg < k_real, 0.0, -100.0).astype(jnp.bfloat16)
        mc_ref[pl.ds(i * KB, KB), :] = jnp.concatenate([codes, guard], axis=1)
        sim = block_sim(i, KB)
        cb_ref[...] += jnp.sum(jnp.where(sim >= 0.0, 1.0, 0.0), axis=1,
                               keepdims=True)

    @pl.when(i == nblk)
    def _phase_b():
        c0 = cb_ref[...]
        ge = c0 >= float(TOPK)
        lo = jnp.where(ge, 0.0, 17.0)
        hi = jnp.where(ge, 16.0, 32.0)
        cb = jnp.where(ge, 0.0, c0)

        def pass_step(_, carry):
            lo, hi, cb = carry
            mid = jnp.floor((lo + hi) * 0.5)
            vmid = float(NBITS) - 2.0 * mid

            def blk(c, acc):
                sim = block_sim(c, KB)
                return acc + jnp.sum(
                    jnp.where(sim >= vmid, 1.0, 0.0), axis=1, keepdims=True)

            c = jax.lax.fori_loop(
                0, nblk, blk, jnp.zeros((qshape, 1), jnp.float32))
            ge = c >= float(TOPK)
            return (jnp.where(ge, lo, mid + 1.0),
                    jnp.where(ge, mid, hi),
                    jnp.where(ge, cb, c))

        lo, hi, cb = jax.lax.fori_loop(0, 5, pass_step, (lo, hi, cb))
        vt_ref[...] = float(NBITS) - 2.0 * hi
        rr_ref[...] = float(TOPK) - cb

    @pl.when(i == nblk + 1)
    def _phase_c():
        vt = vt_ref[...]
        rr = rr_ref[...]
        u = u_ref[...]
        nchunk = (nblk * KB) // CKB

        def sum_step(c, carry):
            acc, runcnt = carry
            sim = block_sim(c, CKB)
            tvrow = tv_ref[:, pl.ds(c * CKB, CKB)]
            acc_gt = jnp.sum(jnp.where(sim > vt, tvrow, 0.0), axis=1,
                             keepdims=True)
            eqf = (sim == vt).astype(jnp.float32)
            pre = jax.lax.dot_general(
                eqf.astype(jnp.bfloat16), u, (((1,), (0,)), ((), ())),
                preferred_element_type=jnp.float32)
            sel = jnp.where(pre + runcnt <= rr, eqf, 0.0)
            acc = acc + acc_gt + jnp.sum(sel * tvrow, axis=1, keepdims=True)
            runcnt = runcnt + jnp.sum(eqf, axis=1, keepdims=True)
            return acc, runcnt

        acc, _ = jax.lax.fori_loop(
            0, nchunk, sum_step,
            (jnp.zeros((qshape, 1), jnp.float32),
             jnp.zeros((qshape, 1), jnp.float32)))
        y_ref[...] = acc * (1.0 / float(TOPK))


@jax.jit
def kernel(h_query, memory_embeds, true_values, R):
    q, d2 = h_query.shape
    k_real = memory_embeds.shape[0]
    nblk = (k_real + KB - 1) // KB
    k_pad = nblk * KB
    if k_pad != k_real:
        memory_embeds = jnp.pad(memory_embeds, ((0, k_pad - k_real), (0, 0)))
        true_values = jnp.pad(true_values, (0, k_pad - k_real))
    tv2 = true_values.reshape(1, k_pad)

    y = pl.pallas_call(
        functools.partial(_body, nblk, k_real),
        grid=(nblk + 2,),
        in_specs=[
            pl.BlockSpec((q, d2), lambda i: (0, 0)),
            pl.BlockSpec((d2, NBITS), lambda i: (0, 0)),
            pl.BlockSpec((KB, d2), lambda i: (jnp.minimum(i, nblk - 1), 0)),
            pl.BlockSpec((1, k_pad), lambda i: (0, 0)),
        ],
        out_specs=pl.BlockSpec((q, 1), lambda i: (0, 0)),
        out_shape=jax.ShapeDtypeStruct((q, 1), jnp.float32),
        scratch_shapes=[
            pltpu.VMEM((q, NCOL), jnp.bfloat16),
            pltpu.VMEM((k_pad, NCOL), jnp.bfloat16),
            pltpu.VMEM((CKB, CKB), jnp.bfloat16),
            pltpu.VMEM((q, 1), jnp.float32),
            pltpu.VMEM((q, 1), jnp.float32),
            pltpu.VMEM((q, 1), jnp.float32),
        ],
        compiler_params=pltpu.CompilerParams(
            dimension_semantics=("arbitrary",)),
    )(h_query, R, memory_embeds, tv2)
    return y[:, 0]


# speedup vs baseline: 1.6329x; 1.0736x over previous
import functools

import jax
import jax.numpy as jnp
from jax.experimental import pallas as pl
from jax.experimental.pallas import tpu as pltpu

NBITS = 32
TOPK = 32
KB = 1024
CKB = 128
NCOL = NBITS + 1


def _body(nblk, k_real, q_ref, r_ref, m_ref, tv_ref, y_ref,
          qc_ref, mc_ref, u_ref, cb_ref, vt_ref, rr_ref):
    i = pl.program_id(0)
    qshape = qc_ref.shape[0]

    def block_sim(c, width):
        mc = mc_ref[pl.ds(c * width, width), :]
        return jax.lax.dot_general(
            qc_ref[...], mc, (((1,), (1,)), ((), ())),
            preferred_element_type=jnp.float32)

    @pl.when(i == 0)
    def _init():
        proj_q = jax.lax.dot_general(
            q_ref[...].astype(jnp.bfloat16), r_ref[...].astype(jnp.bfloat16),
            (((1,), (0,)), ((), ())), preferred_element_type=jnp.float32)
        proj_q = jnp.pad(proj_q, ((0, 0), (0, 1)), constant_values=1.0)
        qc_ref[...] = jnp.where(proj_q > 0, 1.0, -1.0).astype(jnp.bfloat16)
        cb_ref[...] = jnp.zeros_like(cb_ref)
        rowi = jax.lax.broadcasted_iota(jnp.int32, (CKB, CKB), 0)
        coli = jax.lax.broadcasted_iota(jnp.int32, (CKB, CKB), 1)
        u_ref[...] = (rowi <= coli).astype(jnp.bfloat16)

    @pl.when(i < nblk)
    def _phase_a():
        proj_m = jax.lax.dot_general(
            m_ref[...].astype(jnp.bfloat16), r_ref[...].astype(jnp.bfloat16),
            (((1,), (0,)), ((), ())), preferred_element_type=jnp.float32)
        codes = jnp.where(proj_m > 0, 1.0, -1.0).astype(jnp.bfloat16)
        rowg = jax.lax.broadcasted_iota(jnp.int32, (KB, 1), 0) + i * KB
        guard = jnp.where(rowg < k_real, 0.0, -100.0).astype(jnp.bfloat16)
        mc_ref[pl.ds(i * KB, KB), :] = jnp.concatenate([codes, guard], axis=1)
        sim = block_sim(i, KB)
        cols = [jnp.sum(jnp.where(sim >= v, 1.0, 0.0), axis=1, keepdims=True)
                for v in (16.0, 0.0, -16.0)]
        cb_ref[...] += jnp.concatenate(cols, axis=1)

    @pl.when(i == nblk)
    def _phase_b():
        cA = cb_ref[...]
        c8, c16, c24 = cA[:, 0:1], cA[:, 1:2], cA[:, 2:3]
        ge8 = c8 >= float(TOPK)
        ge16 = c16 >= float(TOPK)
        ge24 = c24 >= float(TOPK)
        lo = jnp.where(ge8, 0.0, jnp.where(ge16, 9.0,
                       jnp.where(ge24, 17.0, 25.0)))
        hi = jnp.where(ge8, 8.0, jnp.where(ge16, 16.0,
                       jnp.where(ge24, 24.0, 32.0)))
        cb = jnp.where(ge8, 0.0, jnp.where(ge16, c8,
                       jnp.where(ge24, c16, c24)))

        for _ in range(2):
            r = hi - lo + 1.0
            t1 = lo + jnp.floor((r + 2.0) * (1.0 / 3.0)) - 1.0
            t2 = lo + jnp.floor((2.0 * r + 2.0) * (1.0 / 3.0)) - 1.0
            v1 = float(NBITS) - 2.0 * t1
            v2 = float(NBITS) - 2.0 * t2

            def blk(c, carry):
                a1, a2 = carry
                sim = block_sim(c, KB)
                a1 = a1 + jnp.sum(jnp.where(sim >= v1, 1.0, 0.0), axis=1,
                                  keepdims=True)
                a2 = a2 + jnp.sum(jnp.where(sim >= v2, 1.0, 0.0), axis=1,
                                  keepdims=True)
                return a1, a2

            z = jnp.zeros((qshape, 1), jnp.float32)
            c1, c2 = jax.lax.fori_loop(0, nblk, blk, (z, z))
            ge1 = c1 >= float(TOPK)
            ge2 = c2 >= float(TOPK)
            lo, hi, cb = (
                jnp.where(ge1, lo, jnp.where(ge2, t1 + 1.0, t2 + 1.0)),
                jnp.where(ge1, t1, jnp.where(ge2, t2, hi)),
                jnp.where(ge1, cb, jnp.where(ge2, c1, c2)))

        vt_ref[...] = float(NBITS) - 2.0 * hi
        rr_ref[...] = float(TOPK) - cb

    @pl.when(i == nblk + 1)
    def _phase_c():
        vt = vt_ref[...]
        rr = rr_ref[...]
        u = u_ref[...]
        nchunk = (nblk * KB) // CKB

        def sum_step(c, carry):
            acc, runcnt = carry
            sim = block_sim(c, CKB)
            tvrow = tv_ref[:, pl.ds(c * CKB, CKB)]
            acc_gt = jnp.sum(jnp.where(sim > vt, tvrow, 0.0), axis=1,
                             keepdims=True)
            eqf = (sim == vt).astype(jnp.float32)
            pre = jax.lax.dot_general(
                eqf.astype(jnp.bfloat16), u, (((1,), (0,)), ((), ())),
                preferred_element_type=jnp.float32)
            sel = jnp.where(pre + runcnt <= rr, eqf, 0.0)
            acc = acc + acc_gt + jnp.sum(sel * tvrow, axis=1, keepdims=True)
            runcnt = runcnt + jnp.sum(eqf, axis=1, keepdims=True)
            return acc, runcnt

        acc, _ = jax.lax.fori_loop(
            0, nchunk, sum_step,
            (jnp.zeros((qshape, 1), jnp.float32),
             jnp.zeros((qshape, 1), jnp.float32)))
        y_ref[...] = acc * (1.0 / float(TOPK))


@jax.jit
def kernel(h_query, memory_embeds, true_values, R):
    q, d2 = h_query.shape
    k_real = memory_embeds.shape[0]
    nblk = (k_real + KB - 1) // KB
    k_pad = nblk * KB
    if k_pad != k_real:
        memory_embeds = jnp.pad(memory_embeds, ((0, k_pad - k_real), (0, 0)))
        true_values = jnp.pad(true_values, (0, k_pad - k_real))
    tv2 = true_values.reshape(1, k_pad)

    y = pl.pallas_call(
        functools.partial(_body, nblk, k_real),
        grid=(nblk + 2,),
        in_specs=[
            pl.BlockSpec((q, d2), lambda i: (0, 0)),
            pl.BlockSpec((d2, NBITS), lambda i: (0, 0)),
            pl.BlockSpec((KB, d2), lambda i: (jnp.minimum(i, nblk - 1), 0)),
            pl.BlockSpec((1, k_pad), lambda i: (0, 0)),
        ],
        out_specs=pl.BlockSpec((q, 1), lambda i: (0, 0)),
        out_shape=jax.ShapeDtypeStruct((q, 1), jnp.float32),
        scratch_shapes=[
            pltpu.VMEM((q, NCOL), jnp.bfloat16),
            pltpu.VMEM((k_pad, NCOL), jnp.bfloat16),
            pltpu.VMEM((CKB, CKB), jnp.bfloat16),
            pltpu.VMEM((q, 3), jnp.float32),
            pltpu.VMEM((q, 1), jnp.float32),
            pltpu.VMEM((q, 1), jnp.float32),
        ],
        compiler_params=pltpu.CompilerParams(
            dimension_semantics=("arbitrary",)),
    )(h_query, R, memory_embeds, tv2)
    return y[:, 0]


# speedup vs baseline: 2.0930x; 1.2818x over previous
import functools

import jax
import jax.numpy as jnp
from jax.experimental import pallas as pl
from jax.experimental.pallas import tpu as pltpu

NBITS = 32
TOPK = 32
KB = 1024
CKB = 128
NCOL = NBITS + 1


def _body(nblk, k_real, q_ref, r_ref, m_ref, tv_ref, y_ref,
          qc_ref, mc_ref, u_ref, cb_ref, vt_ref, rr_ref):
    i = pl.program_id(0)
    qshape = qc_ref.shape[0]

    def block_sim(c, width):
        mc = mc_ref[pl.ds(c * width, width), :]
        return jax.lax.dot_general(
            qc_ref[...], mc, (((1,), (1,)), ((), ())),
            preferred_element_type=jnp.float32)

    @pl.when(i == 0)
    def _init():
        proj_q = jax.lax.dot_general(
            q_ref[...].astype(jnp.bfloat16), r_ref[...].astype(jnp.bfloat16),
            (((1,), (0,)), ((), ())), preferred_element_type=jnp.float32)
        proj_q = jnp.pad(proj_q, ((0, 0), (0, 1)), constant_values=1.0)
        qc_ref[...] = jnp.where(proj_q > 0, 1.0, -1.0).astype(jnp.bfloat16)
        cb_ref[...] = jnp.zeros_like(cb_ref)
        rowi = jax.lax.broadcasted_iota(jnp.int32, (CKB, CKB), 0)
        coli = jax.lax.broadcasted_iota(jnp.int32, (CKB, CKB), 1)
        u_ref[...] = (rowi <= coli).astype(jnp.bfloat16)

    @pl.when(i < nblk)
    def _phase_a():
        proj_m = jax.lax.dot_general(
            m_ref[...].astype(jnp.bfloat16), r_ref[...].astype(jnp.bfloat16),
            (((1,), (0,)), ((), ())), preferred_element_type=jnp.float32)
        codes = jnp.where(proj_m > 0, 1.0, -1.0).astype(jnp.bfloat16)
        rowg = jax.lax.broadcasted_iota(jnp.int32, (KB, 1), 0) + i * KB
        guard = jnp.where(rowg < k_real, 0.0, -100.0).astype(jnp.bfloat16)
        mc_ref[pl.ds(i * KB, KB), :] = jnp.concatenate([codes, guard], axis=1)
        sim = block_sim(i, KB)
        cols = [jnp.sum(jnp.where(sim >= v, 1.0, 0.0), axis=1, keepdims=True)
                for v in (16.0, 0.0, -16.0)]
        cb_ref[...] += jnp.concatenate(cols, axis=1)

    @pl.when(i == nblk)
    def _phase_b():
        cA = cb_ref[...]
        c8, c16, c24 = cA[:, 0:1], cA[:, 1:2], cA[:, 2:3]
        ge8 = c8 >= float(TOPK)
        ge16 = c16 >= float(TOPK)
        ge24 = c24 >= float(TOPK)
        lo = jnp.where(ge8, 0.0, jnp.where(ge16, 9.0,
                       jnp.where(ge24, 17.0, 25.0)))
        hi = jnp.where(ge8, 8.0, jnp.where(ge16, 16.0,
                       jnp.where(ge24, 24.0, 32.0)))
        cb = jnp.where(ge8, 0.0, jnp.where(ge16, c8,
                       jnp.where(ge24, c16, c24)))

        for _ in range(2):
            r = hi - lo + 1.0
            t1 = lo + jnp.floor((r + 2.0) * (1.0 / 3.0)) - 1.0
            t2 = lo + jnp.floor((2.0 * r + 2.0) * (1.0 / 3.0)) - 1.0
            v1 = float(NBITS) - 2.0 * t1
            v2 = float(NBITS) - 2.0 * t2

            def blk(c, carry):
                a1, a2 = carry
                sim = block_sim(c, KB)
                a1 = a1 + jnp.sum(jnp.where(sim >= v1, 1.0, 0.0), axis=1,
                                  keepdims=True)
                a2 = a2 + jnp.sum(jnp.where(sim >= v2, 1.0, 0.0), axis=1,
                                  keepdims=True)
                return a1, a2

            z = jnp.zeros((qshape, 1), jnp.float32)
            c1, c2 = jax.lax.fori_loop(0, nblk, blk, (z, z))
            ge1 = c1 >= float(TOPK)
            ge2 = c2 >= float(TOPK)
            lo, hi, cb = (
                jnp.where(ge1, lo, jnp.where(ge2, t1 + 1.0, t2 + 1.0)),
                jnp.where(ge1, t1, jnp.where(ge2, t2, hi)),
                jnp.where(ge1, cb, jnp.where(ge2, c1, c2)))

        vt_ref[...] = float(NBITS) - 2.0 * hi
        rr_ref[...] = float(TOPK) - cb

    @pl.when(i == nblk + 1)
    def _phase_c():
        vt = vt_ref[...]
        rr = rr_ref[...]
        u = u_ref[...]

        def sum_step(c, carry):
            acc, runcnt = carry
            sim = block_sim(c, KB)
            tvrow = tv_ref[:, pl.ds(c * KB, KB)]
            gt = sim > vt
            eq = sim == vt
            eqb = jnp.where(eq, 1.0, 0.0).astype(jnp.bfloat16)
            for s in range(KB // CKB):
                pre = jax.lax.dot_general(
                    eqb[:, s * CKB:(s + 1) * CKB], u,
                    (((1,), (0,)), ((), ())),
                    preferred_element_type=jnp.float32)
                ok = (pre + runcnt <= rr) & eq[:, s * CKB:(s + 1) * CKB]
                keep = ok | gt[:, s * CKB:(s + 1) * CKB]
                acc = acc + jnp.sum(
                    jnp.where(keep, tvrow[:, s * CKB:(s + 1) * CKB], 0.0),
                    axis=1, keepdims=True)
                runcnt = runcnt + pre[:, CKB - 1:CKB]
            return acc, runcnt

        acc, _ = jax.lax.fori_loop(
            0, nblk, sum_step,
            (jnp.zeros((qshape, 1), jnp.float32),
             jnp.zeros((qshape, 1), jnp.float32)))
        y_ref[...] = acc * (1.0 / float(TOPK))


@jax.jit
def kernel(h_query, memory_embeds, true_values, R):
    q, d2 = h_query.shape
    k_real = memory_embeds.shape[0]
    nblk = (k_real + KB - 1) // KB
    k_pad = nblk * KB
    if k_pad != k_real:
        memory_embeds = jnp.pad(memory_embeds, ((0, k_pad - k_real), (0, 0)))
        true_values = jnp.pad(true_values, (0, k_pad - k_real))
    tv2 = true_values.reshape(1, k_pad)

    y = pl.pallas_call(
        functools.partial(_body, nblk, k_real),
        grid=(nblk + 2,),
        in_specs=[
            pl.BlockSpec((q, d2), lambda i: (0, 0)),
            pl.BlockSpec((d2, NBITS), lambda i: (0, 0)),
            pl.BlockSpec((KB, d2), lambda i: (jnp.minimum(i, nblk - 1), 0)),
            pl.BlockSpec((1, k_pad), lambda i: (0, 0)),
        ],
        out_specs=pl.BlockSpec((q, 1), lambda i: (0, 0)),
        out_shape=jax.ShapeDtypeStruct((q, 1), jnp.float32),
        scratch_shapes=[
            pltpu.VMEM((q, NCOL), jnp.bfloat16),
            pltpu.VMEM((k_pad, NCOL), jnp.bfloat16),
            pltpu.VMEM((CKB, CKB), jnp.bfloat16),
            pltpu.VMEM((q, 3), jnp.float32),
            pltpu.VMEM((q, 1), jnp.float32),
            pltpu.VMEM((q, 1), jnp.float32),
        ],
        compiler_params=pltpu.CompilerParams(
            dimension_semantics=("arbitrary",)),
    )(h_query, R, memory_embeds, tv2)
    return y[:, 0]
